# baseline (speedup 1.0000x reference)
import math

import jax
import jax.numpy as jnp
from jax import lax
from jax.experimental import pallas as pl
from jax.experimental.pallas import tpu as pltpu
from jax.experimental.pallas import tpu_sc as plsc

_D = 64
_LANES = 16
_NC, _NS = 2, 16
_NW = _NC * _NS
_IW = 128
_CROWS = 2
_CHUNK = _IW * _CROWS
_NBUF = 4


def kernel(x, table):
    b, h = x.shape
    n = b * h
    scale = jnp.float32(math.sqrt(_D))

    n_rows = n // _IW
    rows_per_w = n_rows // _NW
    nchunks = rows_per_w // _CROWS

    x2 = x.reshape(n_rows, _IW)

    def body(x_hbm, tab_hbm, out_hbm, idx_v, rows_v,
             g0, g1, g2, g3, o0, o1, o2, o3):
        gs = [g0, g1, g2, g3]
        os_ = [o0, o1, o2, o3]
        wid = lax.axis_index("s") * _NC + lax.axis_index("c")
        row0 = wid * rows_per_w
        pltpu.sync_copy(x_hbm.at[pl.ds(row0, rows_per_w)], idx_v)

        def issue_gather(ci, bb):
            for k in range(_CROWS):
                pltpu.async_copy(
                    tab_hbm.at[idx_v.at[ci * _CROWS + k]],
                    rows_v.at[bb, pl.ds(k * _IW, _IW)],
                    gs[bb],
                )

        def drain_gather(bb):
            for k in range(_CROWS):
                pltpu.make_async_copy(
                    out_hbm.at[pl.ds(0, _IW)],
                    rows_v.at[bb, pl.ds(k * _IW, _IW)],
                    gs[bb],
                ).wait()

        def scale_buf(bb):
            def srow(r, c):
                for rr in range(4):
                    for j in range(_D // _LANES):
                        sl = pl.ds(j * _LANES, _LANES)
                        rows_v[bb, r * 4 + rr, sl] = (
                            rows_v[bb, r * 4 + rr, sl] * scale
                        )
                return c

            lax.fori_loop(0, _CHUNK // 4, srow, 0)

        def issue_out(ci, bb):
            pltpu.async_copy(
                rows_v.at[bb],
                out_hbm.at[pl.ds((row0 + ci * _CROWS) * _IW, _CHUNK)],
                os_[bb],
            )

        def drain_out(bb):
            pltpu.make_async_copy(
                rows_v.at[bb],
                out_hbm.at[pl.ds(0, _CHUNK)],
                os_[bb],
            ).wait()

        for bb in range(_NBUF):
            issue_gather(bb, bb)

        def group(gi, c):
            i0 = gi * _NBUF
            for bb in range(_NBUF):
                ci = i0 + bb
                drain_gather(bb)
                scale_buf(bb)
                issue_out(ci, bb)
                drain_out(bb)
                issue_gather(ci + _NBUF, bb)
            return c

        lax.fori_loop(0, nchunks // _NBUF - 1, group, 0)

        i0 = nchunks - _NBUF
        for bb in range(_NBUF):
            drain_gather(bb)
            scale_buf(bb)
            issue_out(i0 + bb, bb)
        for bb in range(_NBUF):
            drain_out(bb)

    out = pl.kernel(
        body,
        out_type=jax.ShapeDtypeStruct((n, _D), jnp.float32),
        mesh=plsc.VectorSubcoreMesh(core_axis_name="c", subcore_axis_name="s"),
        compiler_params=pltpu.CompilerParams(
            use_tc_tiling_on_sc=False, skip_device_barrier=True
        ),
        scratch_types=[
            pltpu.VMEM((rows_per_w, _IW), jnp.int32),
            pltpu.VMEM((_NBUF, _CHUNK, _D), jnp.float32),
        ] + [pltpu.SemaphoreType.DMA] * (2 * _NBUF),
    )(x2, table)

    return out.reshape(b, h, _D)


# speedup vs baseline: 1.0022x; 1.0022x over previous
import math

import jax
import jax.numpy as jnp
from jax import lax
from jax.experimental import pallas as pl
from jax.experimental.pallas import tpu as pltpu
from jax.experimental.pallas import tpu_sc as plsc

_D = 64
_LANES = 16
_NC, _NS = 2, 16
_NW = _NC * _NS
_NBUF = 4


def kernel(x, table):
    nb, h = x.shape
    scale = jnp.float32(math.sqrt(_D))

    b_per_w = nb // _NW
    splits = []
    off = 0
    while off < h:
        w = min(128, h - off)
        splits.append((off, w))
        off += w

    def body(x_hbm, tab_hbm, out_hbm, idx_v, rows_v, g0, g1, g2, g3,
             o0, o1, o2, o3):
        gs = [g0, g1, g2, g3]
        os_ = [o0, o1, o2, o3]
        wid = lax.axis_index("s") * _NC + lax.axis_index("c")
        b0 = wid * b_per_w
        pltpu.sync_copy(x_hbm.at[pl.ds(b0, b_per_w)], idx_v)

        def issue_gather(ci, bb):
            for (o, w) in splits:
                pltpu.async_copy(
                    tab_hbm.at[idx_v.at[ci, pl.ds(o, w)]],
                    rows_v.at[bb, pl.ds(o, w)],
                    gs[bb],
                )

        def drain_gather(bb):
            for (o, w) in splits:
                pltpu.make_async_copy(
                    tab_hbm.at[pl.ds(0, w)],
                    rows_v.at[bb, pl.ds(o, w)],
                    gs[bb],
                ).wait()

        def scale_buf(bb):
            def srow(r, c):
                for rr in range(4):
                    for j in range(_D // _LANES):
                        sl = pl.ds(j * _LANES, _LANES)
                        rows_v[bb, r * 4 + rr, sl] = (
                            rows_v[bb, r * 4 + rr, sl] * scale
                        )
                return c

            lax.fori_loop(0, h // 4, srow, 0)

        def issue_out(ci, bb):
            pltpu.async_copy(rows_v.at[bb], out_hbm.at[b0 + ci], os_[bb])

        def drain_out(bb):
            pltpu.make_async_copy(rows_v.at[bb], out_hbm.at[0], os_[bb]).wait()

        for bb in range(_NBUF):
            issue_gather(bb, bb)

        def group(gi, c):
            i0 = gi * _NBUF
            for bb in range(_NBUF):
                ci = i0 + bb
                drain_gather(bb)
                scale_buf(bb)
                issue_out(ci, bb)
                drain_out(bb)
                issue_gather(ci + _NBUF, bb)
            return c

        lax.fori_loop(0, b_per_w // _NBUF - 1, group, 0)

        i0 = b_per_w - _NBUF
        for bb in range(_NBUF):
            drain_gather(bb)
            scale_buf(bb)
            issue_out(i0 + bb, bb)
        for bb in range(_NBUF):
            drain_out(bb)

    return pl.kernel(
        body,
        out_type=jax.ShapeDtypeStruct((nb, h, _D), jnp.float32),
        mesh=plsc.VectorSubcoreMesh(core_axis_name="c", subcore_axis_name="s"),
        compiler_params=pltpu.CompilerParams(use_tc_tiling_on_sc=False),
        scratch_types=[
            pltpu.VMEM((b_per_w, h), jnp.int32),
            pltpu.VMEM((_NBUF, h, _D), jnp.float32),
        ] + [pltpu.SemaphoreType.DMA] * (2 * _NBUF),
    )(x, table)
